# baseline (speedup 1.0000x reference)
import functools

import jax
import jax.numpy as jnp
from jax import lax
from jax.experimental import pallas as pl
from jax.experimental.pallas import tpu as pltpu
from jax.experimental.pallas import tpu_sc as plsc

_N = 10000
_D = 256
_H = 128
_B = 128
_NB = 80
_EPT = _B * _NB
_E_PAD = 16 * _EPT


def _halfmm_body(x_ref, w_ref, o_ref):
    o_ref[0] = lax.dot_general(
        x_ref[...], w_ref[0], (((1,), (1,)), ((), ())),
        preferred_element_type=jnp.float32,
        precision=lax.Precision.HIGHEST,
    )


def _matmul_halves(x, w):
    n, k = x.shape
    w2 = w.reshape(2, _H, k)
    bm = 1000
    return pl.pallas_call(
        _halfmm_body,
        grid=(n // bm, 2),
        in_specs=[
            pl.BlockSpec((bm, k), lambda i, g: (i, 0)),
            pl.BlockSpec((1, _H, k), lambda i, g: (g, 0, 0)),
        ],
        out_specs=pl.BlockSpec((1, bm, _H), lambda i, g: (g, i, 0)),
        out_shape=jax.ShapeDtypeStruct((2, n, _H), jnp.float32),
    )(x, w2)


def _fuse_body(x_ref, w_ref, b_ref, acc_ref, rs_ref, o_ref):
    y = lax.dot_general(
        x_ref[...], w_ref[0], (((1,), (1,)), ((), ())),
        preferred_element_type=jnp.float32,
        precision=lax.Precision.HIGHEST,
    )
    y = y + b_ref[0] + acc_ref[0]
    y = jnp.maximum(y, 0.0)
    mask = (rs_ref[...] > 0.0).astype(jnp.float32)
    o_ref[...] = y * mask


def _fuse(x, w, b, acc, rowsum):
    n, k = x.shape
    w2 = w.reshape(2, _H, k)
    b2 = b.reshape(2, 1, _H)
    rs = rowsum.reshape(n, 1)
    bm = 1000
    return pl.pallas_call(
        _fuse_body,
        grid=(n // bm, 2),
        in_specs=[
            pl.BlockSpec((bm, k), lambda i, g: (i, 0)),
            pl.BlockSpec((1, _H, k), lambda i, g: (g, 0, 0)),
            pl.BlockSpec((1, 1, _H), lambda i, g: (g, 0, 0)),
            pl.BlockSpec((1, bm, _H), lambda i, g: (g, i, 0)),
            pl.BlockSpec((bm, 1), lambda i, g: (i, 0)),
        ],
        out_specs=pl.BlockSpec((bm, _H), lambda i, g: (i, g)),
        out_shape=jax.ShapeDtypeStruct((n, _D), jnp.float32),
    )(x, w2, b2, acc, rs)


def _sc_spmm(table_u, table_i, rows_u, cols_u, vals_u,
             rows_i, cols_i, vals_i, zeros2d, zeros1d):
    mesh = plsc.VectorSubcoreMesh(core_axis_name="c", subcore_axis_name="s")

    @functools.partial(
        pl.kernel,
        mesh=mesh,
        out_type=[
            jax.ShapeDtypeStruct((2, _N, _H), jnp.float32),
            jax.ShapeDtypeStruct((2, _N, _H), jnp.float32),
            jax.ShapeDtypeStruct((_N,), jnp.float32),
            jax.ShapeDtypeStruct((_N,), jnp.float32),
        ],
        scratch_types=[
            pltpu.VMEM((2, _B), jnp.int32),
            pltpu.VMEM((2, _B), jnp.int32),
            pltpu.VMEM((2, _B), jnp.float32),
            pltpu.VMEM((2, _B, _H), jnp.float32),
            pltpu.VMEM_SHARED((_N, _H), jnp.float32),
            pltpu.VMEM_SHARED((_N,), jnp.float32),
            pltpu.VMEM((1000,), jnp.float32),
            pltpu.SemaphoreType.DMA,
            pltpu.SemaphoreType.DMA,
        ],
    )
    def k(tu_h, ti_h, ru_h, cu_h, vu_h, ri_h, ci_h, vi_h, z2_h, z1_h,
          accu_h, acci_h, rsu_h, rsi_h,
          col_v, row_v, val_v, rows_buf, acc_sp, rs_sp, rs_stage,
          sem0, sem1):
        c = lax.axis_index("c")
        s = lax.axis_index("s")
        gsem = (sem0, sem1)

        def run_pass(table_h, rows_h, cols_h, vals_h, acc_out_h, rs_out_h,
                     rs_core):
            @pl.when(s < 15)
            def _():
                pltpu.sync_copy(z2_h.at[pl.ds(s * 632, 632)],
                                acc_sp.at[pl.ds(s * 632, 632)])

            @pl.when(s == 15)
            def _():
                pltpu.sync_copy(z2_h.at[pl.ds(9480, 520)],
                                acc_sp.at[pl.ds(9480, 520)])

            @pl.when(jnp.logical_and(c == rs_core, s < 10))
            def _():
                pltpu.sync_copy(z1_h.at[pl.ds(s * 1000, 1000)], rs_stage)
                pltpu.sync_copy(rs_stage, rs_sp.at[pl.ds(s * 1000, 1000)])

            plsc.subcore_barrier()

            base = s * _EPT

            def load_idx(b, k):
                off = base + b * _B
                pltpu.sync_copy(cols_h.at[c, pl.ds(off, _B)], col_v.at[k])
                pltpu.sync_copy(rows_h.at[pl.ds(off, _B)], row_v.at[k])
                pltpu.sync_copy(vals_h.at[pl.ds(off, _B)], val_v.at[k])

            def start_gather(k):
                pltpu.make_async_copy(
                    table_h.at[col_v.at[k]], rows_buf.at[k], gsem[k]).start()

            def wait_gather(k):
                pltpu.make_async_copy(
                    table_h.at[col_v.at[k]], rows_buf.at[k], gsem[k]).wait()

            load_idx(0, 0)
            start_gather(0)

            def pair(p, carry):
                for k in range(2):
                    m = p * 2 + k
                    kn = 1 - k
                    wait_gather(k)
                    load_idx(m + 1, kn)
                    start_gather(kn)

                    def scale(g, carry2, k=k):
                        v16 = val_v[k, pl.ds(g * 16, 16)]
                        for l in range(16):
                            e = g * 16 + l
                            v = v16[l]
                            for j in range(_H // 16):
                                slj = pl.ds(j * 16, 16)
                                rows_buf[k, e, slj] = rows_buf[k, e, slj] * v
                        return carry2

                    lax.fori_loop(0, _B // 16, scale, 0)
                    pltpu.sync_copy(rows_buf.at[k], acc_sp.at[row_v.at[k]],
                                    add=True)

                    @pl.when(c == rs_core)
                    def _(k=k):
                        pltpu.sync_copy(val_v.at[k], rs_sp.at[row_v.at[k]],
                                        add=True)

                return carry

            lax.fori_loop(0, _NB // 2, pair, 0)
            wait_gather(_NB % 2)
            plsc.subcore_barrier()

            @pl.when(s < 15)
            def _():
                pltpu.sync_copy(acc_sp.at[pl.ds(s * 632, 632)],
                                acc_out_h.at[c, pl.ds(s * 632, 632)])

            @pl.when(s == 15)
            def _():
                pltpu.sync_copy(acc_sp.at[pl.ds(9480, 520)],
                                acc_out_h.at[c, pl.ds(9480, 520)])

            @pl.when(jnp.logical_and(c == rs_core, s < 10))
            def _():
                pltpu.sync_copy(rs_sp.at[pl.ds(s * 1000, 1000)], rs_stage)
                pltpu.sync_copy(rs_stage, rs_out_h.at[pl.ds(s * 1000, 1000)])

            plsc.subcore_barrier()

        run_pass(tu_h, ru_h, cu_h, vu_h, accu_h, rsu_h, 0)
        run_pass(ti_h, ri_h, ci_h, vi_h, acci_h, rsi_h, 1)

    return k(table_u, table_i, rows_u, cols_u, vals_u,
             rows_i, cols_i, vals_i, zeros2d, zeros1d)


def _pad_edges(index, values):
    rows = index[0].astype(jnp.int32)
    cols = index[1].astype(jnp.int32)
    pad = _E_PAD + _B - rows.shape[0]
    rows = jnp.concatenate([rows, jnp.zeros((pad,), jnp.int32)])
    cols = jnp.concatenate([cols, jnp.zeros((pad,), jnp.int32)])
    vals = jnp.concatenate([values.astype(jnp.float32),
                            jnp.zeros((pad,), jnp.float32)])
    cols2 = jnp.stack([cols, cols + _N])
    return rows, cols2, vals


def kernel(xu_t, xi_t, i2u_index, i2u_values, u2i_index, u2i_values,
           W_uu, b_uu, W_ii, b_ii, W_ui, W_iu):
    iu2 = _matmul_halves(xi_t, W_iu)
    ui2 = _matmul_halves(xu_t, W_ui)
    table_u = iu2.reshape(2 * _N, _H)
    table_i = ui2.reshape(2 * _N, _H)

    ru, cu, vu = _pad_edges(i2u_index, i2u_values)
    ri, ci, vi = _pad_edges(u2i_index, u2i_values)
    z2 = jnp.zeros((_N, _H), jnp.float32)
    z1 = jnp.zeros((_N,), jnp.float32)

    acc_u, acc_i, rs_u, rs_i = _sc_spmm(
        table_u, table_i, ru, cu, vu, ri, ci, vi, z2, z1)

    delta_u = _fuse(xu_t, W_uu, b_uu, acc_u, rs_u)
    delta_i = _fuse(xi_t, W_ii, b_ii, acc_i, rs_i)
    return (delta_u, delta_i)


# speedup vs baseline: 1.1306x; 1.1306x over previous
import functools

import jax
import jax.numpy as jnp
from jax import lax
from jax.experimental import pallas as pl
from jax.experimental.pallas import tpu as pltpu
from jax.experimental.pallas import tpu_sc as plsc

_N = 10000
_D = 256
_H = 128
_B = 128
_NB = 80
_EPT = _B * _NB
_E_PAD = 16 * _EPT
_CH = 40
_NCH = _NB // _CH


def _halfmm_body(x_ref, w_ref, o_ref):
    o_ref[0] = lax.dot_general(
        x_ref[...], w_ref[0], (((1,), (1,)), ((), ())),
        preferred_element_type=jnp.float32,
        precision=lax.Precision.HIGHEST,
    )


def _matmul_halves(x, w):
    n, k = x.shape
    w2 = w.reshape(2, _H, k)
    bm = 1000
    return pl.pallas_call(
        _halfmm_body,
        grid=(n // bm, 2),
        in_specs=[
            pl.BlockSpec((bm, k), lambda i, g: (i, 0)),
            pl.BlockSpec((1, _H, k), lambda i, g: (g, 0, 0)),
        ],
        out_specs=pl.BlockSpec((1, bm, _H), lambda i, g: (g, i, 0)),
        out_shape=jax.ShapeDtypeStruct((2, n, _H), jnp.float32),
    )(x, w2)


def _fuse_body(x_ref, w_ref, b_ref, acc_ref, rs_ref, o_ref):
    y = lax.dot_general(
        x_ref[...], w_ref[0], (((1,), (1,)), ((), ())),
        preferred_element_type=jnp.float32,
        precision=lax.Precision.HIGHEST,
    )
    y = y + b_ref[0] + acc_ref[0]
    y = jnp.maximum(y, 0.0)
    mask = (rs_ref[...] > 0.0).astype(jnp.float32)
    o_ref[...] = y * mask


def _fuse(x, w, b, acc, rowsum):
    n, k = x.shape
    w2 = w.reshape(2, _H, k)
    b2 = b.reshape(2, 1, _H)
    rs = rowsum.reshape(n, 1)
    bm = 1000
    return pl.pallas_call(
        _fuse_body,
        grid=(n // bm, 2),
        in_specs=[
            pl.BlockSpec((bm, k), lambda i, g: (i, 0)),
            pl.BlockSpec((1, _H, k), lambda i, g: (g, 0, 0)),
            pl.BlockSpec((1, 1, _H), lambda i, g: (g, 0, 0)),
            pl.BlockSpec((1, bm, _H), lambda i, g: (g, i, 0)),
            pl.BlockSpec((bm, 1), lambda i, g: (i, 0)),
        ],
        out_specs=pl.BlockSpec((bm, _H), lambda i, g: (i, g)),
        out_shape=jax.ShapeDtypeStruct((n, _D), jnp.float32),
    )(x, w2, b2, acc, rs)


def _sc_spmm(table_u, table_i, rows_u, cols_u, vals_u,
             rows_i, cols_i, vals_i, zeros2d, zeros1d):
    mesh = plsc.VectorSubcoreMesh(core_axis_name="c", subcore_axis_name="s")

    @functools.partial(
        pl.kernel,
        mesh=mesh,
        out_type=[
            jax.ShapeDtypeStruct((2, _N, _H), jnp.float32),
            jax.ShapeDtypeStruct((2, _N, _H), jnp.float32),
            jax.ShapeDtypeStruct((_N,), jnp.float32),
            jax.ShapeDtypeStruct((_N,), jnp.float32),
        ],
        scratch_types=[
            pltpu.VMEM((_CH, _B), jnp.int32),
            pltpu.VMEM((_CH, _B), jnp.int32),
            pltpu.VMEM((_CH, _B), jnp.float32),
            pltpu.VMEM((2, _B, _H), jnp.float32),
            pltpu.VMEM_SHARED((_N, _H), jnp.float32),
            pltpu.VMEM_SHARED((_N,), jnp.float32),
            pltpu.VMEM((1000,), jnp.float32),
            pltpu.SemaphoreType.DMA,
            pltpu.SemaphoreType.DMA,
            pltpu.SemaphoreType.DMA,
            pltpu.SemaphoreType.DMA,
            pltpu.SemaphoreType.DMA,
        ],
    )
    def k(tu_h, ti_h, ru_h, cu_h, vu_h, ri_h, ci_h, vi_h, z2_h, z1_h,
          accu_h, acci_h, rsu_h, rsi_h,
          colb, rowb, valb, rows_buf, acc_sp, rs_sp, rs_stage,
          g0, g1, s0, s1, rsem):
        c = lax.axis_index("c")
        s = lax.axis_index("s")
        gsem = (g0, g1)
        ssem = (s0, s1)

        def run_pass(table_h, rows_h, cols_h, vals_h, acc_out_h, rs_out_h,
                     rs_core):
            @pl.when(s < 15)
            def _():
                pltpu.sync_copy(z2_h.at[pl.ds(s * 632, 632)],
                                acc_sp.at[pl.ds(s * 632, 632)])

            @pl.when(s == 15)
            def _():
                pltpu.sync_copy(z2_h.at[pl.ds(9480, 520)],
                                acc_sp.at[pl.ds(9480, 520)])

            @pl.when(jnp.logical_and(c == rs_core, s < 10))
            def _():
                pltpu.sync_copy(z1_h.at[pl.ds(s * 1000, 1000)], rs_stage)
                pltpu.sync_copy(rs_stage, rs_sp.at[pl.ds(s * 1000, 1000)])

            def load_chunk(ch):
                pltpu.sync_copy(cols_h.at[c, s, ch], colb)
                pltpu.sync_copy(rows_h.at[s, ch], rowb)
                pltpu.sync_copy(vals_h.at[s, ch], valb)

            def start_gather(t, k):
                pltpu.async_copy(
                    table_h.at[colb.at[t]], rows_buf.at[k], gsem[k])

            def wait_gather(t, k):
                pltpu.make_async_copy(
                    table_h.at[colb.at[t]], rows_buf.at[k], gsem[k]).wait()

            def start_scatter(t, k):
                pltpu.async_copy(
                    rows_buf.at[k], acc_sp.at[rowb.at[t]], ssem[k],
                    add=True)

            def wait_scatter(k):
                pltpu.make_async_copy(
                    rows_buf.at[k], acc_sp.at[rowb.at[0]], ssem[k]).wait()

            def drain_rowsum():
                @pl.when(c == rs_core)
                def _():
                    def drain(b, carry):
                        pltpu.make_async_copy(
                            valb.at[0], rs_sp.at[rowb.at[0]], rsem).wait()
                        return carry

                    lax.fori_loop(0, _CH, drain, 0)

            load_chunk(0)
            plsc.subcore_barrier()

            for ch in range(_NCH):
                if ch > 0:
                    wait_scatter((_CH - 1) % 2)
                    drain_rowsum()
                    load_chunk(ch)

                start_gather(0, 0)

                def pair(p, carry):
                    for k in range(2):
                        t = p * 2 + k
                        wait_gather(t, k)

                        def scale(g, carry2, k=k, t=t):
                            v16 = valb[t, pl.ds(g * 16, 16)]
                            for l in range(16):
                                e = g * 16 + l
                                v = v16[l]
                                for j in range(_H // 16):
                                    slj = pl.ds(j * 16, 16)
                                    rows_buf[k, e, slj] = (
                                        rows_buf[k, e, slj] * v)
                            return carry2

                        lax.fori_loop(0, _B // 16, scale, 0)

                        @pl.when(t >= 1)
                        def _(k=k):
                            wait_scatter(1 - k)

                        @pl.when(t + 1 < _CH)
                        def _(t=t, k=k):
                            start_gather(t + 1, 1 - k)

                        start_scatter(t, k)

                        @pl.when(c == rs_core)
                        def _(t=t):
                            pltpu.async_copy(
                                valb.at[t], rs_sp.at[rowb.at[t]], rsem,
                                add=True)

                    return carry

                lax.fori_loop(0, _CH // 2, pair, 0)

            wait_scatter((_CH - 1) % 2)
            drain_rowsum()

            plsc.subcore_barrier()

            @pl.when(s < 15)
            def _():
                pltpu.sync_copy(acc_sp.at[pl.ds(s * 632, 632)],
                                acc_out_h.at[c, pl.ds(s * 632, 632)])

            @pl.when(s == 15)
            def _():
                pltpu.sync_copy(acc_sp.at[pl.ds(9480, 520)],
                                acc_out_h.at[c, pl.ds(9480, 520)])

            @pl.when(jnp.logical_and(c == rs_core, s < 10))
            def _():
                pltpu.sync_copy(rs_sp.at[pl.ds(s * 1000, 1000)], rs_stage)
                pltpu.sync_copy(rs_stage, rs_out_h.at[pl.ds(s * 1000, 1000)])

            plsc.subcore_barrier()

        run_pass(tu_h, ru_h, cu_h, vu_h, accu_h, rsu_h, 0)
        run_pass(ti_h, ri_h, ci_h, vi_h, acci_h, rsi_h, 1)

    return k(table_u, table_i, rows_u, cols_u, vals_u,
             rows_i, cols_i, vals_i, zeros2d, zeros1d)


def _pad_edges(index, values):
    rows = index[0].astype(jnp.int32)
    cols = index[1].astype(jnp.int32)
    pad = _E_PAD - rows.shape[0]
    rows = jnp.concatenate([rows, jnp.zeros((pad,), jnp.int32)])
    cols = jnp.concatenate([cols, jnp.zeros((pad,), jnp.int32)])
    vals = jnp.concatenate([values.astype(jnp.float32),
                            jnp.zeros((pad,), jnp.float32)])
    cols2 = jnp.stack([cols, cols + _N]).reshape(2, 16, _NCH, _CH, _B)
    rows = rows.reshape(16, _NCH, _CH, _B)
    vals = vals.reshape(16, _NCH, _CH, _B)
    return rows, cols2, vals


def kernel(xu_t, xi_t, i2u_index, i2u_values, u2i_index, u2i_values,
           W_uu, b_uu, W_ii, b_ii, W_ui, W_iu):
    iu2 = _matmul_halves(xi_t, W_iu)
    ui2 = _matmul_halves(xu_t, W_ui)
    table_u = iu2.reshape(2 * _N, _H)
    table_i = ui2.reshape(2 * _N, _H)

    ru, cu, vu = _pad_edges(i2u_index, i2u_values)
    ri, ci, vi = _pad_edges(u2i_index, u2i_values)
    z2 = jnp.zeros((_N, _H), jnp.float32)
    z1 = jnp.zeros((_N,), jnp.float32)

    acc_u, acc_i, rs_u, rs_i = _sc_spmm(
        table_u, table_i, ru, cu, vu, ri, ci, vi, z2, z1)

    delta_u = _fuse(xu_t, W_uu, b_uu, acc_u, rs_u)
    delta_i = _fuse(xi_t, W_ii, b_ii, acc_i, rs_i)
    return (delta_u, delta_i)
